# baseline (speedup 1.0000x reference)
import jax
import jax.numpy as jnp
from jax.experimental import pallas as pl
from jax.experimental.pallas import tpu as pltpu

_B = 512
_BLK = 512
_W = 32
_EPS = 1e-6


def _fused_kernel(nblk, idx3_ref, idx2_ref, x_ref, w_ref, b_ref, out_ref,
                  sum_ref, sq_ref, cnt_ref, sc_ref, sh_ref):
    pi = pl.program_id(0)

    @pl.when(pi == 0)
    def _zero():
        sum_ref[...] = jnp.zeros_like(sum_ref)
        sq_ref[...] = jnp.zeros_like(sq_ref)
        cnt_ref[...] = jnp.zeros_like(cnt_ref)

    @pl.when(pi < nblk)
    def _stats():
        x = x_ref[...]
        idx = idx3_ref[0]
        lo = idx3_ref[0, 0, 0]
        hi = idx3_ref[0, 0, _BLK - 1]
        narrow = (hi - lo) < (_W - 8)
        lo_a = jnp.minimum((lo // 8) * 8, _B - _W)

        @pl.when(narrow)
        def _narrow():
            iota = jax.lax.broadcasted_iota(jnp.int32, (_W, 1), 0) + lo_a
            oh = (iota == idx).astype(jnp.float32)
            s = jax.lax.dot_general(oh, x, (((1,), (0,)), ((), ())),
                                    preferred_element_type=jnp.float32)
            q = jax.lax.dot_general(oh, x * x, (((1,), (0,)), ((), ())),
                                    preferred_element_type=jnp.float32)
            sum_ref[pl.ds(lo_a, _W), :] += s
            sq_ref[pl.ds(lo_a, _W), :] += q
            cnt_ref[pl.ds(lo_a, _W), :] += jnp.sum(oh, axis=1, keepdims=True)

        @pl.when(jnp.logical_not(narrow))
        def _wide():
            iota = jax.lax.broadcasted_iota(jnp.int32, (_B, 1), 0)
            oh = (iota == idx).astype(jnp.float32)
            s = jax.lax.dot_general(oh, x, (((1,), (0,)), ((), ())),
                                    preferred_element_type=jnp.float32)
            q = jax.lax.dot_general(oh, x * x, (((1,), (0,)), ((), ())),
                                    preferred_element_type=jnp.float32)
            sum_ref[...] += s
            sq_ref[...] += q
            cnt_ref[...] += jnp.sum(oh, axis=1, keepdims=True)

    @pl.when(pi == nblk)
    def _prep():
        cnt = cnt_ref[...]
        inv = 1.0 / jnp.maximum(cnt, 1.0)
        mean = sum_ref[...] * inv
        var = jnp.maximum(sq_ref[...] * inv - mean * mean, 0.0)
        rstd = jax.lax.rsqrt(var + _EPS)
        scale = w_ref[...] * rstd
        sc_ref[...] = scale
        sh_ref[...] = b_ref[...] - mean * scale

    @pl.when(pi >= nblk)
    def _norm():
        x = x_ref[...]
        idx = idx2_ref[...]
        lo = idx2_ref[0, 0]
        hi = idx2_ref[_BLK - 1, 0]
        narrow = (hi - lo) < (_W - 8)
        lo_a = jnp.minimum((lo // 8) * 8, _B - _W)

        @pl.when(narrow)
        def _narrow():
            iota = jax.lax.broadcasted_iota(jnp.int32, (1, _W), 1) + lo_a
            oh = (idx == iota).astype(jnp.float32)
            gs = jax.lax.dot_general(oh, sc_ref[pl.ds(lo_a, _W), :],
                                     (((1,), (0,)), ((), ())),
                                     preferred_element_type=jnp.float32)
            gt = jax.lax.dot_general(oh, sh_ref[pl.ds(lo_a, _W), :],
                                     (((1,), (0,)), ((), ())),
                                     preferred_element_type=jnp.float32)
            out_ref[...] = x * gs + gt

        @pl.when(jnp.logical_not(narrow))
        def _wide():
            iota = jax.lax.broadcasted_iota(jnp.int32, (1, _B), 1)
            oh = (idx == iota).astype(jnp.float32)
            gs = jax.lax.dot_general(oh, sc_ref[...],
                                     (((1,), (0,)), ((), ())),
                                     preferred_element_type=jnp.float32)
            gt = jax.lax.dot_general(oh, sh_ref[...],
                                     (((1,), (0,)), ((), ())),
                                     preferred_element_type=jnp.float32)
            out_ref[...] = x * gs + gt


def kernel(tensor, weight, bias, batch_index):
    n, d = tensor.shape
    idx = batch_index.astype(jnp.int32)
    nblk = pl.cdiv(n, _BLK)
    npad = nblk * _BLK
    pad = npad - n
    x = jnp.pad(tensor, ((0, pad), (0, 0)))
    idx_p = jnp.pad(idx, (0, pad), constant_values=_B)
    idx3 = idx_p.reshape(nblk, 1, _BLK)
    idx2 = idx_p.reshape(npad, 1)
    w2 = weight.reshape(1, d)
    b2 = bias.reshape(1, d)

    import functools
    body = functools.partial(_fused_kernel, nblk)

    blk_of = lambda i: jnp.where(i < nblk, i, i - nblk)

    out = pl.pallas_call(
        body,
        grid=(2 * nblk,),
        in_specs=[
            pl.BlockSpec((1, 1, _BLK), lambda i: (blk_of(i), 0, 0)),
            pl.BlockSpec((_BLK, 1), lambda i: (blk_of(i), 0)),
            pl.BlockSpec((_BLK, d), lambda i: (blk_of(i), 0)),
            pl.BlockSpec((1, d), lambda i: (0, 0)),
            pl.BlockSpec((1, d), lambda i: (0, 0)),
        ],
        out_specs=pl.BlockSpec((_BLK, d),
                               lambda i: (jnp.where(i < nblk, 0, i - nblk), 0)),
        out_shape=jax.ShapeDtypeStruct((npad, d), jnp.float32),
        scratch_shapes=[
            pltpu.VMEM((_B, d), jnp.float32),
            pltpu.VMEM((_B, d), jnp.float32),
            pltpu.VMEM((_B, 1), jnp.float32),
            pltpu.VMEM((_B, d), jnp.float32),
            pltpu.VMEM((_B, d), jnp.float32),
        ],
    )(idx3, idx2, x, w2, b2)

    return out[:n]


# speedup vs baseline: 1.4503x; 1.4503x over previous
import functools

import jax
import jax.numpy as jnp
from jax.experimental import pallas as pl
from jax.experimental.pallas import tpu as pltpu

_B = 512
_BLK = 1024
_W = 32
_EPS = 1e-6

_TDIMS = (((1,), (0,)), ((), ()))
_GDIMS = (((0,), (0,)), ((), ()))


def _fused_kernel(nblk, idx_ref, x_ref, w_ref, b_ref, out_ref,
                  sum_ref, sq_ref, cnt_ref, sc_ref, sh_ref):
    pi = pl.program_id(0)

    @pl.when(pi == 0)
    def _zero():
        sum_ref[...] = jnp.zeros_like(sum_ref)
        sq_ref[...] = jnp.zeros_like(sq_ref)
        cnt_ref[...] = jnp.zeros_like(cnt_ref)

    idx = idx_ref[0]
    lo = idx_ref[0, 0, 0]
    hi = idx_ref[0, 0, _BLK - 1]
    narrow = (hi - lo) < (_W - 8)
    lo_a = jnp.minimum((lo // 8) * 8, _B - _W)

    @pl.when(pi < nblk)
    def _stats():
        xb = x_ref[...].astype(jnp.bfloat16)

        @pl.when(narrow)
        def _narrow():
            iota = jax.lax.broadcasted_iota(jnp.int32, (_W, 1), 0) + lo_a
            oh = (iota == idx).astype(jnp.bfloat16)
            s = jax.lax.dot_general(oh, xb, _TDIMS,
                                    preferred_element_type=jnp.float32)
            q = jax.lax.dot_general(oh, xb * xb, _TDIMS,
                                    preferred_element_type=jnp.float32)
            sum_ref[pl.ds(lo_a, _W), :] += s
            sq_ref[pl.ds(lo_a, _W), :] += q
            cnt_ref[pl.ds(lo_a, _W), :] += jnp.sum(
                oh.astype(jnp.float32), axis=1, keepdims=True)

        @pl.when(jnp.logical_not(narrow))
        def _wide():
            iota = jax.lax.broadcasted_iota(jnp.int32, (_B, 1), 0)
            oh = (iota == idx).astype(jnp.bfloat16)
            s = jax.lax.dot_general(oh, xb, _TDIMS,
                                    preferred_element_type=jnp.float32)
            q = jax.lax.dot_general(oh, xb * xb, _TDIMS,
                                    preferred_element_type=jnp.float32)
            sum_ref[...] += s
            sq_ref[...] += q
            cnt_ref[...] += jnp.sum(
                oh.astype(jnp.float32), axis=1, keepdims=True)

    @pl.when(pi == nblk)
    def _prep():
        cnt = cnt_ref[...]
        inv = 1.0 / jnp.maximum(cnt, 1.0)
        mean = sum_ref[...] * inv
        var = jnp.maximum(sq_ref[...] * inv - mean * mean, 0.0)
        rstd = jax.lax.rsqrt(var + _EPS)
        scale = w_ref[...] * rstd
        sc_ref[...] = scale.astype(jnp.bfloat16)
        sh_ref[...] = (b_ref[...] - mean * scale).astype(jnp.bfloat16)

    @pl.when(pi >= nblk)
    def _norm():
        x = x_ref[...]

        @pl.when(narrow)
        def _narrow():
            iota = jax.lax.broadcasted_iota(jnp.int32, (_W, 1), 0) + lo_a
            oh = (iota == idx).astype(jnp.bfloat16)
            gs = jax.lax.dot_general(oh, sc_ref[pl.ds(lo_a, _W), :], _GDIMS,
                                     preferred_element_type=jnp.float32)
            gt = jax.lax.dot_general(oh, sh_ref[pl.ds(lo_a, _W), :], _GDIMS,
                                     preferred_element_type=jnp.float32)
            out_ref[...] = x * gs + gt

        @pl.when(jnp.logical_not(narrow))
        def _wide():
            iota = jax.lax.broadcasted_iota(jnp.int32, (_B, 1), 0)
            oh = (iota == idx).astype(jnp.bfloat16)
            gs = jax.lax.dot_general(oh, sc_ref[...], _GDIMS,
                                     preferred_element_type=jnp.float32)
            gt = jax.lax.dot_general(oh, sh_ref[...], _GDIMS,
                                     preferred_element_type=jnp.float32)
            out_ref[...] = x * gs + gt


def kernel(tensor, weight, bias, batch_index):
    n, d = tensor.shape
    idx = batch_index.astype(jnp.int32)
    nblk = pl.cdiv(n, _BLK)
    npad = nblk * _BLK
    pad = npad - n
    x = jnp.pad(tensor, ((0, pad), (0, 0)))
    idx_p = jnp.pad(idx, (0, pad), constant_values=_B)
    idx3 = idx_p.reshape(nblk, 1, _BLK)
    w2 = weight.reshape(1, d)
    b2 = bias.reshape(1, d)

    body = functools.partial(_fused_kernel, nblk)
    blk_of = lambda i: jnp.where(i < nblk, i, i - nblk)

    out = pl.pallas_call(
        body,
        grid=(2 * nblk,),
        in_specs=[
            pl.BlockSpec((1, 1, _BLK), lambda i: (blk_of(i), 0, 0)),
            pl.BlockSpec((_BLK, d), lambda i: (blk_of(i), 0)),
            pl.BlockSpec((1, d), lambda i: (0, 0)),
            pl.BlockSpec((1, d), lambda i: (0, 0)),
        ],
        out_specs=pl.BlockSpec((_BLK, d),
                               lambda i: (jnp.where(i < nblk, 0, i - nblk), 0)),
        out_shape=jax.ShapeDtypeStruct((npad, d), jnp.float32),
        scratch_shapes=[
            pltpu.VMEM((_B, d), jnp.float32),
            pltpu.VMEM((_B, d), jnp.float32),
            pltpu.VMEM((_B, 1), jnp.float32),
            pltpu.VMEM((_B, d), jnp.bfloat16),
            pltpu.VMEM((_B, d), jnp.bfloat16),
        ],
    )(idx3, x, w2, b2)

    return out[:n]


# speedup vs baseline: 1.6097x; 1.1099x over previous
import functools

import jax
import jax.numpy as jnp
from jax.experimental import pallas as pl
from jax.experimental.pallas import tpu as pltpu

_B = 512
_BLK = 1024
_W = 64
_EPS = 1e-6

_TDIMS = (((1,), (0,)), ((), ()))
_GDIMS = (((0,), (0,)), ((), ()))


def _fused_kernel(nblk, idx_ref, x_ref, w_ref, b_ref, out_ref,
                  sum_ref, sq_ref, cnt_ref, sc_ref, sh_ref, xc_ref):
    pi = pl.program_id(0)

    @pl.when(pi == 0)
    def _zero():
        sum_ref[...] = jnp.zeros_like(sum_ref)
        sq_ref[...] = jnp.zeros_like(sq_ref)
        cnt_ref[...] = jnp.zeros_like(cnt_ref)

    idx = idx_ref[0]
    lo = idx_ref[0, 0, 0]
    hi = idx_ref[0, 0, _BLK - 1]
    narrow = (hi - lo) < (_W - 16)
    lo_a = jnp.minimum((lo // 16) * 16, _B - _W)

    @pl.when(pi < nblk)
    def _stats():
        xb = x_ref[...].astype(jnp.bfloat16)
        xc_ref[pl.ds(pi * _BLK, _BLK), :] = xb

        @pl.when(narrow)
        def _narrow():
            iota = jax.lax.broadcasted_iota(jnp.int32, (_W, 1), 0) + lo_a
            oh = (iota == idx).astype(jnp.bfloat16)
            s = jax.lax.dot_general(oh, xb, _TDIMS,
                                    preferred_element_type=jnp.float32)
            q = jax.lax.dot_general(oh, xb * xb, _TDIMS,
                                    preferred_element_type=jnp.float32)
            sum_ref[pl.ds(lo_a, _W), :] += s
            sq_ref[pl.ds(lo_a, _W), :] += q
            cnt_ref[pl.ds(lo_a, _W), :] += jnp.sum(
                oh.astype(jnp.float32), axis=1, keepdims=True)

        @pl.when(jnp.logical_not(narrow))
        def _wide():
            iota = jax.lax.broadcasted_iota(jnp.int32, (_B, 1), 0)
            oh = (iota == idx).astype(jnp.bfloat16)
            s = jax.lax.dot_general(oh, xb, _TDIMS,
                                    preferred_element_type=jnp.float32)
            q = jax.lax.dot_general(oh, xb * xb, _TDIMS,
                                    preferred_element_type=jnp.float32)
            sum_ref[...] += s
            sq_ref[...] += q
            cnt_ref[...] += jnp.sum(
                oh.astype(jnp.float32), axis=1, keepdims=True)

    @pl.when(pi == nblk)
    def _prep():
        cnt = cnt_ref[...]
        inv = 1.0 / jnp.maximum(cnt, 1.0)
        mean = sum_ref[...] * inv
        var = jnp.maximum(sq_ref[...] * inv - mean * mean, 0.0)
        rstd = jax.lax.rsqrt(var + _EPS)
        scale = w_ref[...] * rstd
        sc_ref[...] = scale.astype(jnp.bfloat16)
        sh_ref[...] = (b_ref[...] - mean * scale).astype(jnp.bfloat16)

    @pl.when(pi >= nblk)
    def _norm():
        x = xc_ref[pl.ds((pi - nblk) * _BLK, _BLK), :].astype(jnp.float32)

        @pl.when(narrow)
        def _narrow():
            iota = jax.lax.broadcasted_iota(jnp.int32, (_W, 1), 0) + lo_a
            oh = (iota == idx).astype(jnp.bfloat16)
            gs = jax.lax.dot_general(oh, sc_ref[pl.ds(lo_a, _W), :], _GDIMS,
                                     preferred_element_type=jnp.float32)
            gt = jax.lax.dot_general(oh, sh_ref[pl.ds(lo_a, _W), :], _GDIMS,
                                     preferred_element_type=jnp.float32)
            out_ref[...] = x * gs + gt

        @pl.when(jnp.logical_not(narrow))
        def _wide():
            iota = jax.lax.broadcasted_iota(jnp.int32, (_B, 1), 0)
            oh = (iota == idx).astype(jnp.bfloat16)
            gs = jax.lax.dot_general(oh, sc_ref[...], _GDIMS,
                                     preferred_element_type=jnp.float32)
            gt = jax.lax.dot_general(oh, sh_ref[...], _GDIMS,
                                     preferred_element_type=jnp.float32)
            out_ref[...] = x * gs + gt


def kernel(tensor, weight, bias, batch_index):
    n, d = tensor.shape
    idx = batch_index.astype(jnp.int32)
    nblk = pl.cdiv(n, _BLK)
    npad = nblk * _BLK
    pad = npad - n
    x = jnp.pad(tensor, ((0, pad), (0, 0)))
    idx_p = jnp.pad(idx, (0, pad), constant_values=_B)
    idx3 = idx_p.reshape(nblk, 1, _BLK)
    w2 = weight.reshape(1, d)
    b2 = bias.reshape(1, d)

    body = functools.partial(_fused_kernel, nblk)
    blk_of = lambda i: jnp.where(i < nblk, i, i - nblk)

    out = pl.pallas_call(
        body,
        grid=(2 * nblk,),
        in_specs=[
            pl.BlockSpec((1, 1, _BLK), lambda i: (blk_of(i), 0, 0)),
            pl.BlockSpec((_BLK, d), lambda i: (jnp.where(i < nblk, i, 0), 0)),
            pl.BlockSpec((1, d), lambda i: (0, 0)),
            pl.BlockSpec((1, d), lambda i: (0, 0)),
        ],
        out_specs=pl.BlockSpec((_BLK, d),
                               lambda i: (jnp.where(i < nblk, 0, i - nblk), 0)),
        out_shape=jax.ShapeDtypeStruct((npad, d), jnp.float32),
        scratch_shapes=[
            pltpu.VMEM((_B, d), jnp.float32),
            pltpu.VMEM((_B, d), jnp.float32),
            pltpu.VMEM((_B, 1), jnp.float32),
            pltpu.VMEM((_B, d), jnp.bfloat16),
            pltpu.VMEM((_B, d), jnp.bfloat16),
            pltpu.VMEM((npad, d), jnp.bfloat16),
        ],
    )(idx3, x, w2, b2)

    return out[:n]


# speedup vs baseline: 1.9398x; 1.2051x over previous
import functools

import jax
import jax.numpy as jnp
from jax.experimental import pallas as pl
from jax.experimental.pallas import tpu as pltpu

_B = 512
_BLK = 2048
_W = 64
_EPS = 1e-6

_TDIMS = (((1,), (0,)), ((), ()))
_GDIMS = (((0,), (0,)), ((), ()))


def _fused_kernel(nblk, idx_ref, x_ref, w_ref, b_ref, out_ref,
                  sum_ref, sq_ref, cnt_ref, sc_ref, sh_ref, xc_ref):
    pi = pl.program_id(0)

    @pl.when(pi == 0)
    def _zero():
        sum_ref[...] = jnp.zeros_like(sum_ref)
        sq_ref[...] = jnp.zeros_like(sq_ref)
        cnt_ref[...] = jnp.zeros_like(cnt_ref)

    idx = idx_ref[0]
    lo = idx_ref[0, 0, 0]
    hi = idx_ref[0, 0, _BLK - 1]
    narrow = (hi - lo) < (_W - 16)
    lo_a = jnp.minimum((lo // 16) * 16, _B - _W)

    @pl.when(pi < nblk)
    def _stats():
        xb = x_ref[...].astype(jnp.bfloat16)
        xc_ref[pl.ds(pi * _BLK, _BLK), :] = xb

        @pl.when(narrow)
        def _narrow():
            iota = jax.lax.broadcasted_iota(jnp.int32, (_W, 1), 0) + lo_a
            oh = (iota == idx).astype(jnp.bfloat16)
            s = jax.lax.dot_general(oh, xb, _TDIMS,
                                    preferred_element_type=jnp.float32)
            q = jax.lax.dot_general(oh, xb * xb, _TDIMS,
                                    preferred_element_type=jnp.float32)
            sum_ref[pl.ds(lo_a, _W), :] += s
            sq_ref[pl.ds(lo_a, _W), :] += q
            cnt_ref[pl.ds(lo_a, _W), :] += jnp.sum(
                oh.astype(jnp.float32), axis=1, keepdims=True)

        @pl.when(jnp.logical_not(narrow))
        def _wide():
            iota = jax.lax.broadcasted_iota(jnp.int32, (_B, 1), 0)
            oh = (iota == idx).astype(jnp.bfloat16)
            s = jax.lax.dot_general(oh, xb, _TDIMS,
                                    preferred_element_type=jnp.float32)
            q = jax.lax.dot_general(oh, xb * xb, _TDIMS,
                                    preferred_element_type=jnp.float32)
            sum_ref[...] += s
            sq_ref[...] += q
            cnt_ref[...] += jnp.sum(
                oh.astype(jnp.float32), axis=1, keepdims=True)

    @pl.when(pi == nblk)
    def _prep():
        cnt = cnt_ref[...]
        inv = 1.0 / jnp.maximum(cnt, 1.0)
        mean = sum_ref[...] * inv
        var = jnp.maximum(sq_ref[...] * inv - mean * mean, 0.0)
        rstd = jax.lax.rsqrt(var + _EPS)
        scale = w_ref[...] * rstd
        sc_ref[...] = scale.astype(jnp.bfloat16)
        sh_ref[...] = (b_ref[...] - mean * scale).astype(jnp.bfloat16)

    @pl.when(pi >= nblk)
    def _norm():
        x = xc_ref[pl.ds((pi - nblk) * _BLK, _BLK), :].astype(jnp.float32)

        @pl.when(narrow)
        def _narrow():
            iota = jax.lax.broadcasted_iota(jnp.int32, (_W, 1), 0) + lo_a
            oh = (iota == idx).astype(jnp.bfloat16)
            gs = jax.lax.dot_general(oh, sc_ref[pl.ds(lo_a, _W), :], _GDIMS,
                                     preferred_element_type=jnp.float32)
            gt = jax.lax.dot_general(oh, sh_ref[pl.ds(lo_a, _W), :], _GDIMS,
                                     preferred_element_type=jnp.float32)
            out_ref[...] = x * gs + gt

        @pl.when(jnp.logical_not(narrow))
        def _wide():
            iota = jax.lax.broadcasted_iota(jnp.int32, (_B, 1), 0)
            oh = (iota == idx).astype(jnp.bfloat16)
            gs = jax.lax.dot_general(oh, sc_ref[...], _GDIMS,
                                     preferred_element_type=jnp.float32)
            gt = jax.lax.dot_general(oh, sh_ref[...], _GDIMS,
                                     preferred_element_type=jnp.float32)
            out_ref[...] = x * gs + gt


def kernel(tensor, weight, bias, batch_index):
    n, d = tensor.shape
    idx = batch_index.astype(jnp.int32)
    nblk = pl.cdiv(n, _BLK)
    npad = nblk * _BLK
    pad = npad - n
    x = jnp.pad(tensor, ((0, pad), (0, 0)))
    idx_p = jnp.pad(idx, (0, pad), constant_values=_B)
    idx3 = idx_p.reshape(nblk, 1, _BLK)
    w2 = weight.reshape(1, d)
    b2 = bias.reshape(1, d)

    body = functools.partial(_fused_kernel, nblk)
    blk_of = lambda i: jnp.where(i < nblk, i, i - nblk)

    out = pl.pallas_call(
        body,
        grid=(2 * nblk,),
        in_specs=[
            pl.BlockSpec((1, 1, _BLK), lambda i: (blk_of(i), 0, 0)),
            pl.BlockSpec((_BLK, d), lambda i: (jnp.where(i < nblk, i, 0), 0)),
            pl.BlockSpec((1, d), lambda i: (0, 0)),
            pl.BlockSpec((1, d), lambda i: (0, 0)),
        ],
        out_specs=pl.BlockSpec((_BLK, d),
                               lambda i: (jnp.where(i < nblk, 0, i - nblk), 0)),
        out_shape=jax.ShapeDtypeStruct((npad, d), jnp.float32),
        scratch_shapes=[
            pltpu.VMEM((_B, d), jnp.float32),
            pltpu.VMEM((_B, d), jnp.float32),
            pltpu.VMEM((_B, 1), jnp.float32),
            pltpu.VMEM((_B, d), jnp.bfloat16),
            pltpu.VMEM((_B, d), jnp.bfloat16),
            pltpu.VMEM((npad, d), jnp.bfloat16),
        ],
    )(idx3, x, w2, b2)

    return out[:n]


# speedup vs baseline: 2.1385x; 1.1024x over previous
import functools

import jax
import jax.numpy as jnp
from jax.experimental import pallas as pl
from jax.experimental.pallas import tpu as pltpu

_B = 512
_BLK = 4096
_W = 128
_EPS = 1e-6

_TDIMS = (((1,), (0,)), ((), ()))
_GDIMS = (((0,), (0,)), ((), ()))


def _fused_kernel(nblk, idx_ref, x_ref, w_ref, b_ref, out_ref,
                  sum_ref, sq_ref, cnt_ref, sc_ref, sh_ref, xc_ref):
    pi = pl.program_id(0)

    @pl.when(pi == 0)
    def _zero():
        sum_ref[...] = jnp.zeros_like(sum_ref)
        sq_ref[...] = jnp.zeros_like(sq_ref)
        cnt_ref[...] = jnp.zeros_like(cnt_ref)

    idx = idx_ref[0]
    lo = idx_ref[0, 0, 0]
    hi = idx_ref[0, 0, _BLK - 1]
    narrow = (hi - lo) < (_W - 16)
    lo_a = jnp.minimum((lo // 16) * 16, _B - _W)

    @pl.when(pi < nblk)
    def _stats():
        xb = x_ref[...].astype(jnp.bfloat16)
        xc_ref[pl.ds(pi * _BLK, _BLK), :] = xb

        @pl.when(narrow)
        def _narrow():
            iota = jax.lax.broadcasted_iota(jnp.int32, (_W, 1), 0) + lo_a
            oh = (iota == idx).astype(jnp.bfloat16)
            s = jax.lax.dot_general(oh, xb, _TDIMS,
                                    preferred_element_type=jnp.float32)
            q = jax.lax.dot_general(oh, xb * xb, _TDIMS,
                                    preferred_element_type=jnp.float32)
            sum_ref[pl.ds(lo_a, _W), :] += s
            sq_ref[pl.ds(lo_a, _W), :] += q
            cnt_ref[pl.ds(lo_a, _W), :] += jnp.sum(
                oh.astype(jnp.float32), axis=1, keepdims=True)

        @pl.when(jnp.logical_not(narrow))
        def _wide():
            iota = jax.lax.broadcasted_iota(jnp.int32, (_B, 1), 0)
            oh = (iota == idx).astype(jnp.bfloat16)
            s = jax.lax.dot_general(oh, xb, _TDIMS,
                                    preferred_element_type=jnp.float32)
            q = jax.lax.dot_general(oh, xb * xb, _TDIMS,
                                    preferred_element_type=jnp.float32)
            sum_ref[...] += s
            sq_ref[...] += q
            cnt_ref[...] += jnp.sum(
                oh.astype(jnp.float32), axis=1, keepdims=True)

    @pl.when(pi == nblk)
    def _prep():
        cnt = cnt_ref[...]
        inv = 1.0 / jnp.maximum(cnt, 1.0)
        mean = sum_ref[...] * inv
        var = jnp.maximum(sq_ref[...] * inv - mean * mean, 0.0)
        rstd = jax.lax.rsqrt(var + _EPS)
        scale = w_ref[...] * rstd
        sc_ref[...] = scale.astype(jnp.bfloat16)
        sh_ref[...] = (b_ref[...] - mean * scale).astype(jnp.bfloat16)

    @pl.when(pi >= nblk)
    def _norm():
        x = xc_ref[pl.ds((pi - nblk) * _BLK, _BLK), :].astype(jnp.float32)

        @pl.when(narrow)
        def _narrow():
            iota = jax.lax.broadcasted_iota(jnp.int32, (_W, 1), 0) + lo_a
            oh = (iota == idx).astype(jnp.bfloat16)
            gs = jax.lax.dot_general(oh, sc_ref[pl.ds(lo_a, _W), :], _GDIMS,
                                     preferred_element_type=jnp.float32)
            gt = jax.lax.dot_general(oh, sh_ref[pl.ds(lo_a, _W), :], _GDIMS,
                                     preferred_element_type=jnp.float32)
            out_ref[...] = x * gs + gt

        @pl.when(jnp.logical_not(narrow))
        def _wide():
            iota = jax.lax.broadcasted_iota(jnp.int32, (_B, 1), 0)
            oh = (iota == idx).astype(jnp.bfloat16)
            gs = jax.lax.dot_general(oh, sc_ref[...], _GDIMS,
                                     preferred_element_type=jnp.float32)
            gt = jax.lax.dot_general(oh, sh_ref[...], _GDIMS,
                                     preferred_element_type=jnp.float32)
            out_ref[...] = x * gs + gt


def kernel(tensor, weight, bias, batch_index):
    n, d = tensor.shape
    idx = batch_index.astype(jnp.int32)
    nblk = pl.cdiv(n, _BLK)
    npad = nblk * _BLK
    pad = npad - n
    x = jnp.pad(tensor, ((0, pad), (0, 0)))
    idx_p = jnp.pad(idx, (0, pad), constant_values=_B)
    idx3 = idx_p.reshape(nblk, 1, _BLK)
    w2 = weight.reshape(1, d)
    b2 = bias.reshape(1, d)

    body = functools.partial(_fused_kernel, nblk)
    blk_of = lambda i: jnp.where(i < nblk, i, i - nblk)

    out = pl.pallas_call(
        body,
        grid=(2 * nblk,),
        in_specs=[
            pl.BlockSpec((1, 1, _BLK), lambda i: (blk_of(i), 0, 0)),
            pl.BlockSpec((_BLK, d), lambda i: (jnp.where(i < nblk, i, 0), 0)),
            pl.BlockSpec((1, d), lambda i: (0, 0)),
            pl.BlockSpec((1, d), lambda i: (0, 0)),
        ],
        out_specs=pl.BlockSpec((_BLK, d),
                               lambda i: (jnp.where(i < nblk, 0, i - nblk), 0)),
        out_shape=jax.ShapeDtypeStruct((npad, d), jnp.float32),
        scratch_shapes=[
            pltpu.VMEM((_B, d), jnp.float32),
            pltpu.VMEM((_B, d), jnp.float32),
            pltpu.VMEM((_B, 1), jnp.float32),
            pltpu.VMEM((_B, d), jnp.bfloat16),
            pltpu.VMEM((_B, d), jnp.bfloat16),
            pltpu.VMEM((npad, d), jnp.bfloat16),
        ],
    )(idx3, x, w2, b2)

    return out[:n]


# speedup vs baseline: 2.1819x; 1.0203x over previous
import functools

import jax
import jax.numpy as jnp
from jax.experimental import pallas as pl
from jax.experimental.pallas import tpu as pltpu

_B = 512
_BLK = 5120
_W = 192
_EPS = 1e-6

_TDIMS = (((1,), (0,)), ((), ()))
_GDIMS = (((0,), (0,)), ((), ()))


def _fused_kernel(nblk, idx_ref, x_ref, w_ref, b_ref, out_ref,
                  sum_ref, sq_ref, cnt_ref, sc_ref, sh_ref, xc_ref):
    pi = pl.program_id(0)

    @pl.when(pi == 0)
    def _zero():
        sum_ref[...] = jnp.zeros_like(sum_ref)
        sq_ref[...] = jnp.zeros_like(sq_ref)
        cnt_ref[...] = jnp.zeros_like(cnt_ref)

    idx = idx_ref[0]
    lo = idx_ref[0, 0, 0]
    hi = idx_ref[0, 0, _BLK - 1]
    narrow = (hi - lo) < (_W - 16)
    lo_a = jnp.minimum((lo // 16) * 16, _B - _W)

    @pl.when(pi < nblk)
    def _stats():
        xb = x_ref[...].astype(jnp.bfloat16)
        xc_ref[pl.ds(pi * _BLK, _BLK), :] = xb

        @pl.when(narrow)
        def _narrow():
            iota = jax.lax.broadcasted_iota(jnp.int32, (_W, 1), 0) + lo_a
            oh = (iota == idx).astype(jnp.bfloat16)
            s = jax.lax.dot_general(oh, xb, _TDIMS,
                                    preferred_element_type=jnp.float32)
            q = jax.lax.dot_general(oh, xb * xb, _TDIMS,
                                    preferred_element_type=jnp.float32)
            sum_ref[pl.ds(lo_a, _W), :] += s
            sq_ref[pl.ds(lo_a, _W), :] += q
            cnt_ref[pl.ds(lo_a, _W), :] += jnp.sum(
                oh.astype(jnp.float32), axis=1, keepdims=True)

        @pl.when(jnp.logical_not(narrow))
        def _wide():
            iota = jax.lax.broadcasted_iota(jnp.int32, (_B, 1), 0)
            oh = (iota == idx).astype(jnp.bfloat16)
            s = jax.lax.dot_general(oh, xb, _TDIMS,
                                    preferred_element_type=jnp.float32)
            q = jax.lax.dot_general(oh, xb * xb, _TDIMS,
                                    preferred_element_type=jnp.float32)
            sum_ref[...] += s
            sq_ref[...] += q
            cnt_ref[...] += jnp.sum(
                oh.astype(jnp.float32), axis=1, keepdims=True)

    @pl.when(pi == nblk)
    def _prep():
        cnt = cnt_ref[...]
        inv = 1.0 / jnp.maximum(cnt, 1.0)
        mean = sum_ref[...] * inv
        var = jnp.maximum(sq_ref[...] * inv - mean * mean, 0.0)
        rstd = jax.lax.rsqrt(var + _EPS)
        scale = w_ref[...] * rstd
        sc_ref[...] = scale.astype(jnp.bfloat16)
        sh_ref[...] = (b_ref[...] - mean * scale).astype(jnp.bfloat16)

    @pl.when(pi >= nblk)
    def _norm():
        x = xc_ref[pl.ds((pi - nblk) * _BLK, _BLK), :].astype(jnp.float32)

        @pl.when(narrow)
        def _narrow():
            iota = jax.lax.broadcasted_iota(jnp.int32, (_W, 1), 0) + lo_a
            oh = (iota == idx).astype(jnp.bfloat16)
            gs = jax.lax.dot_general(oh, sc_ref[pl.ds(lo_a, _W), :], _GDIMS,
                                     preferred_element_type=jnp.float32)
            gt = jax.lax.dot_general(oh, sh_ref[pl.ds(lo_a, _W), :], _GDIMS,
                                     preferred_element_type=jnp.float32)
            out_ref[...] = x * gs + gt

        @pl.when(jnp.logical_not(narrow))
        def _wide():
            iota = jax.lax.broadcasted_iota(jnp.int32, (_B, 1), 0)
            oh = (iota == idx).astype(jnp.bfloat16)
            gs = jax.lax.dot_general(oh, sc_ref[...], _GDIMS,
                                     preferred_element_type=jnp.float32)
            gt = jax.lax.dot_general(oh, sh_ref[...], _GDIMS,
                                     preferred_element_type=jnp.float32)
            out_ref[...] = x * gs + gt


def kernel(tensor, weight, bias, batch_index):
    n, d = tensor.shape
    idx = batch_index.astype(jnp.int32)
    nblk = pl.cdiv(n, _BLK)
    npad = nblk * _BLK
    pad = npad - n
    x = jnp.pad(tensor, ((0, pad), (0, 0)))
    idx_p = jnp.pad(idx, (0, pad), constant_values=_B)
    idx3 = idx_p.reshape(nblk, 1, _BLK)
    w2 = weight.reshape(1, d)
    b2 = bias.reshape(1, d)

    body = functools.partial(_fused_kernel, nblk)
    blk_of = lambda i: jnp.where(i < nblk, i, i - nblk)

    out = pl.pallas_call(
        body,
        grid=(2 * nblk,),
        in_specs=[
            pl.BlockSpec((1, 1, _BLK), lambda i: (blk_of(i), 0, 0)),
            pl.BlockSpec((_BLK, d), lambda i: (jnp.where(i < nblk, i, 0), 0)),
            pl.BlockSpec((1, d), lambda i: (0, 0)),
            pl.BlockSpec((1, d), lambda i: (0, 0)),
        ],
        out_specs=pl.BlockSpec((_BLK, d),
                               lambda i: (jnp.where(i < nblk, 0, i - nblk), 0)),
        out_shape=jax.ShapeDtypeStruct((npad, d), jnp.float32),
        scratch_shapes=[
            pltpu.VMEM((_B, d), jnp.float32),
            pltpu.VMEM((_B, d), jnp.float32),
            pltpu.VMEM((_B, 1), jnp.float32),
            pltpu.VMEM((_B, d), jnp.bfloat16),
            pltpu.VMEM((_B, d), jnp.bfloat16),
            pltpu.VMEM((npad, d), jnp.bfloat16),
        ],
    )(idx3, x, w2, b2)

    return out[:n]
